# baseline (speedup 1.0000x reference)
import jax
import jax.numpy as jnp
from jax import lax
from jax.experimental import pallas as pl
from jax.experimental.pallas import tpu as pltpu
from jax.experimental.pallas import tpu_sc as plsc

N_ENTITIES = 1000000
N_RELATIONS = 1000
K = 64
BATCH = 16384

NC = 2
NS = 16
NW = NC * NS
B_PER_W = BATCH // NW
CHUNK = 128
N_CHUNKS = B_PER_W // CHUNK
GROUPS = CHUNK // 16

SPLIT = 512000
TB = 512

_GATHER_DNUMS = lax.GatherDimensionNumbers(
    offset_dims=(), collapsed_slice_dims=(0,), start_index_map=(0,)
)


def _shuffle(v, perm):
    return lax.gather(
        v, perm[:, None], _GATHER_DNUMS, (1,),
        mode=lax.GatherScatterMode.PROMISE_IN_BOUNDS,
    )


def _hsum(v, lane):
    for s in (8, 4, 2, 1):
        v = v + _shuffle(v, lane ^ s)
    return v


def _sqrt16(a):
    a = jnp.maximum(a, jnp.float32(1e-30))
    bits = lax.bitcast_convert_type(a, jnp.int32)
    y = lax.bitcast_convert_type(
        jnp.int32(0x5F3759DF) - lax.shift_right_logical(bits, 1), jnp.float32
    )
    half = jnp.float32(0.5) * a
    for _ in range(3):
        y = y * (jnp.float32(1.5) - half * y * y)
    return a * y


def _transpose_body(lo_ref, hi_ref, out_ref):
    out_ref[:, 0:K] = lo_ref[...].T
    out_ref[:, K:2 * K] = hi_ref[...].T


def _pair_table(ent_t):
    grid = SPLIT // TB
    return pl.pallas_call(
        _transpose_body,
        grid=(grid,),
        in_specs=[
            pl.BlockSpec((K, TB), lambda i: (0, i)),
            pl.BlockSpec(
                (K, TB),
                lambda i: (0, jnp.minimum(i + SPLIT // TB,
                                          N_ENTITIES // TB)),
            ),
        ],
        out_specs=pl.BlockSpec((TB, 2 * K), lambda i: (i, 0)),
        out_shape=jax.ShapeDtypeStruct((SPLIT, 2 * K), jnp.float32),
    )(ent_t, ent_t)


def _sc_body(idx_hbm, ent_hbm, rn_hbm, out_hbm,
             idx_v, hbuf, tbuf, rnbuf, loss_v, s0, s1, s2):
    wid = lax.axis_index("s") * NC + lax.axis_index("c")
    pltpu.sync_copy(idx_hbm.at[wid], idx_v)

    for c in range(N_CHUNKS):
        cp0 = pltpu.async_copy(ent_hbm.at[idx_v.at[c]], hbuf, s0)
        cp1 = pltpu.async_copy(ent_hbm.at[idx_v.at[N_CHUNKS + c]], tbuf, s1)
        cp2 = pltpu.async_copy(rn_hbm.at[idx_v.at[2 * N_CHUNKS + c]], rnbuf, s2)
        cp0.wait()
        cp1.wait()
        cp2.wait()

        def group(g, _):
            lane = lax.iota(jnp.int32, 16)
            acc = jnp.zeros((16,), jnp.float32)
            base = g * 16
            offs_h = idx_v[3 * N_CHUNKS + c, pl.ds(base, 16)]
            offs_t = idx_v[4 * N_CHUNKS + c, pl.ds(base, 16)]
            for i in range(16):
                s = base + i
                oh = offs_h[i]
                ot = offs_t[i]
                h = [hbuf[s, pl.ds(oh + 16 * j, 16)] for j in range(4)]
                t = [tbuf[s, pl.ds(ot + 16 * j, 16)] for j in range(4)]
                r = [rnbuf[s, pl.ds(16 * j, 16)] for j in range(4)]
                n = [rnbuf[s, pl.ds(64 + 16 * j, 16)] for j in range(4)]
                u = [h[j] - t[j] for j in range(4)]
                p = u[0] * n[0] + u[1] * n[1] + u[2] * n[2] + u[3] * n[3]
                d = _hsum(p, lane)
                df = [u[j] + r[j] - d * n[j] for j in range(4)]
                sq = df[0] * df[0] + df[1] * df[1] + df[2] * df[2] + df[3] * df[3]
                ss = _hsum(sq, lane)
                acc = jnp.where(lane == i, ss, acc)
            loss_v[pl.ds(c * CHUNK + g * 16, 16)] = _sqrt16(acc)
            return _

        lax.fori_loop(0, GROUPS, group, 0)

    pltpu.sync_copy(loss_v, out_hbm.at[pl.ds(wid * B_PER_W, B_PER_W)])


@jax.jit
def _transh(idx_pack, ent_t, rel_norm):
    ent2 = lax.optimization_barrier(_pair_table(ent_t))
    mesh = plsc.VectorSubcoreMesh(core_axis_name="c", subcore_axis_name="s")
    kfn = pl.kernel(
        _sc_body,
        out_type=jax.ShapeDtypeStruct((BATCH,), jnp.float32),
        mesh=mesh,
        scratch_types=[
            pltpu.VMEM((5 * N_CHUNKS, CHUNK), jnp.int32),
            pltpu.VMEM((CHUNK, 2 * K), jnp.float32),
            pltpu.VMEM((CHUNK, 2 * K), jnp.float32),
            pltpu.VMEM((CHUNK, 2 * K), jnp.float32),
            pltpu.VMEM((B_PER_W,), jnp.float32),
            pltpu.SemaphoreType.DMA,
            pltpu.SemaphoreType.DMA,
            pltpu.SemaphoreType.DMA,
        ],
    )
    return kfn(idx_pack, ent2, rel_norm)


def kernel(head, relation, tail, entity_emb, relation_emb, norm_emb):
    head = jnp.asarray(head, jnp.int32)
    tail = jnp.asarray(tail, jnp.int32)
    rel = jnp.asarray(relation, jnp.int32).reshape(NW, N_CHUNKS, CHUNK)
    h_line = jnp.where(head < SPLIT, head, head - SPLIT).reshape(NW, N_CHUNKS, CHUNK)
    t_line = jnp.where(tail < SPLIT, tail, tail - SPLIT).reshape(NW, N_CHUNKS, CHUNK)
    h_off = jnp.where(head < SPLIT, 0, K).astype(jnp.int32).reshape(NW, N_CHUNKS, CHUNK)
    t_off = jnp.where(tail < SPLIT, 0, K).astype(jnp.int32).reshape(NW, N_CHUNKS, CHUNK)
    idx_pack = jnp.concatenate([h_line, t_line, rel, h_off, t_off], axis=1)
    rel_norm = jnp.concatenate([relation_emb, norm_emb], axis=1)
    return _transh(idx_pack, entity_emb.T, rel_norm)


# speedup vs baseline: 1.9994x; 1.9994x over previous
import jax
import jax.numpy as jnp
from jax import lax
from jax.experimental import pallas as pl
from jax.experimental.pallas import tpu as pltpu
from jax.experimental.pallas import tpu_sc as plsc

N_ENTITIES = 1000000
N_RELATIONS = 1000
K = 64
BATCH = 16384

NC = 2
NS = 16
NW = NC * NS
B_PER_W = BATCH // NW
CHUNK = 128
N_CHUNKS = B_PER_W // CHUNK
GROUPS = CHUNK // 16

SPLIT = 512000
TB = 2048

_GATHER_DNUMS = lax.GatherDimensionNumbers(
    offset_dims=(), collapsed_slice_dims=(0,), start_index_map=(0,)
)


def _shuffle(v, perm):
    return lax.gather(
        v, perm[:, None], _GATHER_DNUMS, (1,),
        mode=lax.GatherScatterMode.PROMISE_IN_BOUNDS,
    )


def _hsum(v, lane):
    for s in (8, 4, 2, 1):
        v = v + _shuffle(v, lane ^ s)
    return v


def _sqrt16(a):
    a = jnp.maximum(a, jnp.float32(1e-30))
    bits = lax.bitcast_convert_type(a, jnp.int32)
    y = lax.bitcast_convert_type(
        jnp.int32(0x5F3759DF) - lax.shift_right_logical(bits, 1), jnp.float32
    )
    half = jnp.float32(0.5) * a
    for _ in range(3):
        y = y * (jnp.float32(1.5) - half * y * y)
    return a * y


def _transpose_body(lo_ref, hi_ref, out_ref):
    eye = jnp.eye(K, dtype=jnp.float32)
    dn = (((0,), (0,)), ((), ()))
    out_ref[:, 0:K] = lax.dot_general(
        lo_ref[...], eye, dn, preferred_element_type=jnp.float32)
    out_ref[:, K:2 * K] = lax.dot_general(
        hi_ref[...], eye, dn, preferred_element_type=jnp.float32)


def _pair_table(ent_t):
    grid = SPLIT // TB
    return pl.pallas_call(
        _transpose_body,
        grid=(grid,),
        in_specs=[
            pl.BlockSpec((K, TB), lambda i: (0, i)),
            pl.BlockSpec(
                (K, TB),
                lambda i: (0, jnp.minimum(i + SPLIT // TB,
                                          N_ENTITIES // TB)),
            ),
        ],
        out_specs=pl.BlockSpec((TB, 2 * K), lambda i: (i, 0)),
        out_shape=jax.ShapeDtypeStruct((SPLIT, 2 * K), jnp.float32),
    )(ent_t, ent_t)


def _sc_body(idx_hbm, ent_hbm, rn_hbm, out_hbm,
             idx_v, hbuf, tbuf, rnbuf, loss_v, s0, s1, s2):
    wid = lax.axis_index("s") * NC + lax.axis_index("c")
    pltpu.sync_copy(idx_hbm.at[wid], idx_v)

    for c in range(N_CHUNKS):
        cp0 = pltpu.async_copy(ent_hbm.at[idx_v.at[c]], hbuf, s0)
        cp1 = pltpu.async_copy(ent_hbm.at[idx_v.at[N_CHUNKS + c]], tbuf, s1)
        cp2 = pltpu.async_copy(rn_hbm.at[idx_v.at[2 * N_CHUNKS + c]], rnbuf, s2)
        cp0.wait()
        cp1.wait()
        cp2.wait()

        def group(g, _):
            lane = lax.iota(jnp.int32, 16)
            acc = jnp.zeros((16,), jnp.float32)
            base = g * 16
            offs_h = idx_v[3 * N_CHUNKS + c, pl.ds(base, 16)]
            offs_t = idx_v[4 * N_CHUNKS + c, pl.ds(base, 16)]
            for i in range(16):
                s = base + i
                oh = offs_h[i]
                ot = offs_t[i]
                h = [hbuf[s, pl.ds(oh + 16 * j, 16)] for j in range(4)]
                t = [tbuf[s, pl.ds(ot + 16 * j, 16)] for j in range(4)]
                r = [rnbuf[s, pl.ds(16 * j, 16)] for j in range(4)]
                n = [rnbuf[s, pl.ds(64 + 16 * j, 16)] for j in range(4)]
                u = [h[j] - t[j] for j in range(4)]
                p = u[0] * n[0] + u[1] * n[1] + u[2] * n[2] + u[3] * n[3]
                d = _hsum(p, lane)
                df = [u[j] + r[j] - d * n[j] for j in range(4)]
                sq = df[0] * df[0] + df[1] * df[1] + df[2] * df[2] + df[3] * df[3]
                ss = _hsum(sq, lane)
                acc = jnp.where(lane == i, ss, acc)
            loss_v[pl.ds(c * CHUNK + g * 16, 16)] = _sqrt16(acc)
            return _

        lax.fori_loop(0, GROUPS, group, 0)

    pltpu.sync_copy(loss_v, out_hbm.at[pl.ds(wid * B_PER_W, B_PER_W)])


@jax.jit
def _transh(idx_pack, ent_t, rel_norm):
    ent2 = lax.optimization_barrier(_pair_table(ent_t))
    mesh = plsc.VectorSubcoreMesh(core_axis_name="c", subcore_axis_name="s")
    kfn = pl.kernel(
        _sc_body,
        out_type=jax.ShapeDtypeStruct((BATCH,), jnp.float32),
        mesh=mesh,
        scratch_types=[
            pltpu.VMEM((5 * N_CHUNKS, CHUNK), jnp.int32),
            pltpu.VMEM((CHUNK, 2 * K), jnp.float32),
            pltpu.VMEM((CHUNK, 2 * K), jnp.float32),
            pltpu.VMEM((CHUNK, 2 * K), jnp.float32),
            pltpu.VMEM((B_PER_W,), jnp.float32),
            pltpu.SemaphoreType.DMA,
            pltpu.SemaphoreType.DMA,
            pltpu.SemaphoreType.DMA,
        ],
    )
    return kfn(idx_pack, ent2, rel_norm)


def kernel(head, relation, tail, entity_emb, relation_emb, norm_emb):
    head = jnp.asarray(head, jnp.int32)
    tail = jnp.asarray(tail, jnp.int32)
    rel = jnp.asarray(relation, jnp.int32).reshape(NW, N_CHUNKS, CHUNK)
    h_line = jnp.where(head < SPLIT, head, head - SPLIT).reshape(NW, N_CHUNKS, CHUNK)
    t_line = jnp.where(tail < SPLIT, tail, tail - SPLIT).reshape(NW, N_CHUNKS, CHUNK)
    h_off = jnp.where(head < SPLIT, 0, K).astype(jnp.int32).reshape(NW, N_CHUNKS, CHUNK)
    t_off = jnp.where(tail < SPLIT, 0, K).astype(jnp.int32).reshape(NW, N_CHUNKS, CHUNK)
    idx_pack = jnp.concatenate([h_line, t_line, rel, h_off, t_off], axis=1)
    rel_norm = jnp.concatenate([relation_emb, norm_emb], axis=1)
    return _transh(idx_pack, entity_emb.T, rel_norm)


# speedup vs baseline: 2.4431x; 1.2219x over previous
import jax
import jax.numpy as jnp
from jax import lax
from jax.experimental import pallas as pl
from jax.experimental.pallas import tpu as pltpu
from jax.experimental.pallas import tpu_sc as plsc

N_ENTITIES = 1000000
N_RELATIONS = 1000
K = 64
BATCH = 16384

NC = 2
NS = 16
NW = NC * NS
B_PER_W = BATCH // NW
CHUNK = 128
N_CHUNKS = B_PER_W // CHUNK
GROUPS = CHUNK // 16

SPLIT = 512000
TB = 4096

_GATHER_DNUMS = lax.GatherDimensionNumbers(
    offset_dims=(), collapsed_slice_dims=(0,), start_index_map=(0,)
)


def _shuffle(v, perm):
    return lax.gather(
        v, perm[:, None], _GATHER_DNUMS, (1,),
        mode=lax.GatherScatterMode.PROMISE_IN_BOUNDS,
    )


def _hsum(v, lane):
    for s in (8, 4, 2, 1):
        v = v + _shuffle(v, lane ^ s)
    return v


def _sqrt16(a):
    a = jnp.maximum(a, jnp.float32(1e-30))
    bits = lax.bitcast_convert_type(a, jnp.int32)
    y = lax.bitcast_convert_type(
        jnp.int32(0x5F3759DF) - lax.shift_right_logical(bits, 1), jnp.float32
    )
    half = jnp.float32(0.5) * a
    for _ in range(3):
        y = y * (jnp.float32(1.5) - half * y * y)
    return a * y


def _transpose_body(lo_ref, hi_ref, out_ref):
    eye = jnp.eye(K, dtype=jnp.float32)
    dn = (((0,), (0,)), ((), ()))
    out_ref[:, 0:K] = lax.dot_general(
        lo_ref[...], eye, dn, preferred_element_type=jnp.float32)
    out_ref[:, K:2 * K] = lax.dot_general(
        hi_ref[...], eye, dn, preferred_element_type=jnp.float32)


def _pair_table(ent_t):
    grid = SPLIT // TB
    return pl.pallas_call(
        _transpose_body,
        grid=(grid,),
        in_specs=[
            pl.BlockSpec((K, TB), lambda i: (0, i)),
            pl.BlockSpec(
                (K, TB),
                lambda i: (0, jnp.minimum(i + SPLIT // TB,
                                          N_ENTITIES // TB)),
            ),
        ],
        out_specs=pl.BlockSpec((TB, 2 * K), lambda i: (i, 0)),
        out_shape=jax.ShapeDtypeStruct((SPLIT, 2 * K), jnp.float32),
    )(ent_t, ent_t)


def _sc_body(idx_hbm, ent_hbm, rn_hbm, out_hbm,
             idx_v, hbuf, tbuf, rnbuf, loss_v, s0, s1, s2):
    wid = lax.axis_index("s") * NC + lax.axis_index("c")
    pltpu.sync_copy(idx_hbm.at[wid], idx_v)

    for c in range(N_CHUNKS):
        cp0 = pltpu.async_copy(ent_hbm.at[idx_v.at[c]], hbuf, s0)
        cp1 = pltpu.async_copy(ent_hbm.at[idx_v.at[N_CHUNKS + c]], tbuf, s1)
        cp2 = pltpu.async_copy(rn_hbm.at[idx_v.at[2 * N_CHUNKS + c]], rnbuf, s2)
        cp0.wait()
        cp1.wait()
        cp2.wait()

        def group(g, _):
            lane = lax.iota(jnp.int32, 16)
            acc = jnp.zeros((16,), jnp.float32)
            base = g * 16
            offs_h = idx_v[3 * N_CHUNKS + c, pl.ds(base, 16)]
            offs_t = idx_v[4 * N_CHUNKS + c, pl.ds(base, 16)]
            for i in range(16):
                s = base + i
                oh = offs_h[i]
                ot = offs_t[i]
                h = [hbuf[s, pl.ds(oh + 16 * j, 16)] for j in range(4)]
                t = [tbuf[s, pl.ds(ot + 16 * j, 16)] for j in range(4)]
                r = [rnbuf[s, pl.ds(16 * j, 16)] for j in range(4)]
                n = [rnbuf[s, pl.ds(64 + 16 * j, 16)] for j in range(4)]
                u = [h[j] - t[j] for j in range(4)]
                p = u[0] * n[0] + u[1] * n[1] + u[2] * n[2] + u[3] * n[3]
                d = _hsum(p, lane)
                df = [u[j] + r[j] - d * n[j] for j in range(4)]
                sq = df[0] * df[0] + df[1] * df[1] + df[2] * df[2] + df[3] * df[3]
                ss = _hsum(sq, lane)
                acc = jnp.where(lane == i, ss, acc)
            loss_v[pl.ds(c * CHUNK + g * 16, 16)] = _sqrt16(acc)
            return _

        lax.fori_loop(0, GROUPS, group, 0)

    pltpu.sync_copy(loss_v, out_hbm.at[pl.ds(wid * B_PER_W, B_PER_W)])


@jax.jit
def _transh(idx_pack, ent_t, rel_norm):
    ent2 = lax.optimization_barrier(_pair_table(ent_t))
    mesh = plsc.VectorSubcoreMesh(core_axis_name="c", subcore_axis_name="s")
    kfn = pl.kernel(
        _sc_body,
        out_type=jax.ShapeDtypeStruct((BATCH,), jnp.float32),
        mesh=mesh,
        scratch_types=[
            pltpu.VMEM((5 * N_CHUNKS, CHUNK), jnp.int32),
            pltpu.VMEM((CHUNK, 2 * K), jnp.float32),
            pltpu.VMEM((CHUNK, 2 * K), jnp.float32),
            pltpu.VMEM((CHUNK, 2 * K), jnp.float32),
            pltpu.VMEM((B_PER_W,), jnp.float32),
            pltpu.SemaphoreType.DMA,
            pltpu.SemaphoreType.DMA,
            pltpu.SemaphoreType.DMA,
        ],
    )
    return kfn(idx_pack, ent2, rel_norm)


def kernel(head, relation, tail, entity_emb, relation_emb, norm_emb):
    head = jnp.asarray(head, jnp.int32)
    tail = jnp.asarray(tail, jnp.int32)
    rel = jnp.asarray(relation, jnp.int32).reshape(NW, N_CHUNKS, CHUNK)
    h_line = jnp.where(head < SPLIT, head, head - SPLIT).reshape(NW, N_CHUNKS, CHUNK)
    t_line = jnp.where(tail < SPLIT, tail, tail - SPLIT).reshape(NW, N_CHUNKS, CHUNK)
    h_off = jnp.where(head < SPLIT, 0, K).astype(jnp.int32).reshape(NW, N_CHUNKS, CHUNK)
    t_off = jnp.where(tail < SPLIT, 0, K).astype(jnp.int32).reshape(NW, N_CHUNKS, CHUNK)
    idx_pack = jnp.concatenate([h_line, t_line, rel, h_off, t_off], axis=1)
    rel_norm = jnp.concatenate([relation_emb, norm_emb], axis=1)
    return _transh(idx_pack, entity_emb.T, rel_norm)


# speedup vs baseline: 2.8755x; 1.1770x over previous
import jax
import jax.numpy as jnp
from jax import lax
from jax.experimental import pallas as pl
from jax.experimental.pallas import tpu as pltpu
from jax.experimental.pallas import tpu_sc as plsc

N_ENTITIES = 1000000
N_RELATIONS = 1000
K = 64
BATCH = 16384

NC = 2
NS = 16
NW = NC * NS
B_PER_W = BATCH // NW
CHUNK = 128
N_CHUNKS = B_PER_W // CHUNK
GROUPS = CHUNK // 16

SPLIT = 512000
TB = 12800

_GATHER_DNUMS = lax.GatherDimensionNumbers(
    offset_dims=(), collapsed_slice_dims=(0,), start_index_map=(0,)
)


def _shuffle(v, perm):
    return lax.gather(
        v, perm[:, None], _GATHER_DNUMS, (1,),
        mode=lax.GatherScatterMode.PROMISE_IN_BOUNDS,
    )


def _hsum(v, lane):
    for s in (8, 4, 2, 1):
        v = v + _shuffle(v, lane ^ s)
    return v


def _sqrt16(a):
    a = jnp.maximum(a, jnp.float32(1e-30))
    bits = lax.bitcast_convert_type(a, jnp.int32)
    y = lax.bitcast_convert_type(
        jnp.int32(0x5F3759DF) - lax.shift_right_logical(bits, 1), jnp.float32
    )
    half = jnp.float32(0.5) * a
    for _ in range(3):
        y = y * (jnp.float32(1.5) - half * y * y)
    return a * y


def _transpose_body(lo_ref, hi_ref, out_ref):
    eye = jnp.eye(K, dtype=jnp.float32)
    dn = (((0,), (0,)), ((), ()))
    out_ref[:, 0:K] = lax.dot_general(
        lo_ref[...], eye, dn, preferred_element_type=jnp.float32)
    out_ref[:, K:2 * K] = lax.dot_general(
        hi_ref[...], eye, dn, preferred_element_type=jnp.float32)


def _pair_table(ent_t):
    grid = SPLIT // TB
    return pl.pallas_call(
        _transpose_body,
        grid=(grid,),
        in_specs=[
            pl.BlockSpec((K, TB), lambda i: (0, i)),
            pl.BlockSpec(
                (K, TB),
                lambda i: (0, jnp.minimum(i + SPLIT // TB,
                                          N_ENTITIES // TB)),
            ),
        ],
        out_specs=pl.BlockSpec((TB, 2 * K), lambda i: (i, 0)),
        out_shape=jax.ShapeDtypeStruct((SPLIT, 2 * K), jnp.float32),
    )(ent_t, ent_t)


def _sc_body(idx_hbm, ent_hbm, rn_hbm, out_hbm,
             idx_v, hbuf, tbuf, rnbuf, loss_v, s0, s1, s2):
    wid = lax.axis_index("s") * NC + lax.axis_index("c")
    pltpu.sync_copy(idx_hbm.at[wid], idx_v)

    for c in range(N_CHUNKS):
        cp0 = pltpu.async_copy(ent_hbm.at[idx_v.at[c]], hbuf, s0)
        cp1 = pltpu.async_copy(ent_hbm.at[idx_v.at[N_CHUNKS + c]], tbuf, s1)
        cp2 = pltpu.async_copy(rn_hbm.at[idx_v.at[2 * N_CHUNKS + c]], rnbuf, s2)
        cp0.wait()
        cp1.wait()
        cp2.wait()

        def group(g, _):
            lane = lax.iota(jnp.int32, 16)
            acc = jnp.zeros((16,), jnp.float32)
            base = g * 16
            offs_h = idx_v[3 * N_CHUNKS + c, pl.ds(base, 16)]
            offs_t = idx_v[4 * N_CHUNKS + c, pl.ds(base, 16)]
            for i in range(16):
                s = base + i
                oh = offs_h[i]
                ot = offs_t[i]
                h = [hbuf[s, pl.ds(oh + 16 * j, 16)] for j in range(4)]
                t = [tbuf[s, pl.ds(ot + 16 * j, 16)] for j in range(4)]
                r = [rnbuf[s, pl.ds(16 * j, 16)] for j in range(4)]
                n = [rnbuf[s, pl.ds(64 + 16 * j, 16)] for j in range(4)]
                u = [h[j] - t[j] for j in range(4)]
                p = u[0] * n[0] + u[1] * n[1] + u[2] * n[2] + u[3] * n[3]
                d = _hsum(p, lane)
                df = [u[j] + r[j] - d * n[j] for j in range(4)]
                sq = df[0] * df[0] + df[1] * df[1] + df[2] * df[2] + df[3] * df[3]
                ss = _hsum(sq, lane)
                acc = jnp.where(lane == i, ss, acc)
            loss_v[pl.ds(c * CHUNK + g * 16, 16)] = _sqrt16(acc)
            return _

        lax.fori_loop(0, GROUPS, group, 0)

    pltpu.sync_copy(loss_v, out_hbm.at[pl.ds(wid * B_PER_W, B_PER_W)])


@jax.jit
def _transh(idx_pack, ent_t, rel_norm):
    ent2 = lax.optimization_barrier(_pair_table(ent_t))
    mesh = plsc.VectorSubcoreMesh(core_axis_name="c", subcore_axis_name="s")
    kfn = pl.kernel(
        _sc_body,
        out_type=jax.ShapeDtypeStruct((BATCH,), jnp.float32),
        mesh=mesh,
        scratch_types=[
            pltpu.VMEM((5 * N_CHUNKS, CHUNK), jnp.int32),
            pltpu.VMEM((CHUNK, 2 * K), jnp.float32),
            pltpu.VMEM((CHUNK, 2 * K), jnp.float32),
            pltpu.VMEM((CHUNK, 2 * K), jnp.float32),
            pltpu.VMEM((B_PER_W,), jnp.float32),
            pltpu.SemaphoreType.DMA,
            pltpu.SemaphoreType.DMA,
            pltpu.SemaphoreType.DMA,
        ],
    )
    return kfn(idx_pack, ent2, rel_norm)


def kernel(head, relation, tail, entity_emb, relation_emb, norm_emb):
    head = jnp.asarray(head, jnp.int32)
    tail = jnp.asarray(tail, jnp.int32)
    rel = jnp.asarray(relation, jnp.int32).reshape(NW, N_CHUNKS, CHUNK)
    h_line = jnp.where(head < SPLIT, head, head - SPLIT).reshape(NW, N_CHUNKS, CHUNK)
    t_line = jnp.where(tail < SPLIT, tail, tail - SPLIT).reshape(NW, N_CHUNKS, CHUNK)
    h_off = jnp.where(head < SPLIT, 0, K).astype(jnp.int32).reshape(NW, N_CHUNKS, CHUNK)
    t_off = jnp.where(tail < SPLIT, 0, K).astype(jnp.int32).reshape(NW, N_CHUNKS, CHUNK)
    idx_pack = jnp.concatenate([h_line, t_line, rel, h_off, t_off], axis=1)
    rel_norm = jnp.concatenate([relation_emb, norm_emb], axis=1)
    return _transh(idx_pack, entity_emb.T, rel_norm)


# speedup vs baseline: 2.8939x; 1.0064x over previous
import jax
import jax.numpy as jnp
from jax import lax
from jax.experimental import pallas as pl
from jax.experimental.pallas import tpu as pltpu
from jax.experimental.pallas import tpu_sc as plsc

N_ENTITIES = 1000000
N_RELATIONS = 1000
K = 64
BATCH = 16384

NC = 2
NS = 16
NW = NC * NS
B_PER_W = BATCH // NW
CHUNK = 128
N_CHUNKS = B_PER_W // CHUNK
GROUPS = CHUNK // 16

SPLIT = 512000
TB = 20480

_GATHER_DNUMS = lax.GatherDimensionNumbers(
    offset_dims=(), collapsed_slice_dims=(0,), start_index_map=(0,)
)


def _shuffle(v, perm):
    return lax.gather(
        v, perm[:, None], _GATHER_DNUMS, (1,),
        mode=lax.GatherScatterMode.PROMISE_IN_BOUNDS,
    )


def _hsum(v, lane):
    for s in (8, 4, 2, 1):
        v = v + _shuffle(v, lane ^ s)
    return v


def _sqrt16(a):
    a = jnp.maximum(a, jnp.float32(1e-30))
    bits = lax.bitcast_convert_type(a, jnp.int32)
    y = lax.bitcast_convert_type(
        jnp.int32(0x5F3759DF) - lax.shift_right_logical(bits, 1), jnp.float32
    )
    half = jnp.float32(0.5) * a
    for _ in range(3):
        y = y * (jnp.float32(1.5) - half * y * y)
    return a * y


def _transpose_body(lo_ref, hi_ref, out_ref):
    eye = jnp.eye(K, dtype=jnp.float32)
    dn = (((0,), (0,)), ((), ()))
    out_ref[:, 0:K] = lax.dot_general(
        lo_ref[...], eye, dn, preferred_element_type=jnp.float32)
    out_ref[:, K:2 * K] = lax.dot_general(
        hi_ref[...], eye, dn, preferred_element_type=jnp.float32)


def _pair_table(ent_t):
    grid = SPLIT // TB
    return pl.pallas_call(
        _transpose_body,
        grid=(grid,),
        in_specs=[
            pl.BlockSpec((K, TB), lambda i: (0, i)),
            pl.BlockSpec(
                (K, TB),
                lambda i: (0, jnp.minimum(i + SPLIT // TB,
                                          N_ENTITIES // TB)),
            ),
        ],
        out_specs=pl.BlockSpec((TB, 2 * K), lambda i: (i, 0)),
        out_shape=jax.ShapeDtypeStruct((SPLIT, 2 * K), jnp.float32),
    )(ent_t, ent_t)


def _sc_body(idx_hbm, ent_hbm, rn_hbm, out_hbm,
             idx_v, hbuf, tbuf, rnbuf, loss_v, s0, s1, s2):
    wid = lax.axis_index("s") * NC + lax.axis_index("c")
    pltpu.sync_copy(idx_hbm.at[wid], idx_v)

    for c in range(N_CHUNKS):
        cp0 = pltpu.async_copy(ent_hbm.at[idx_v.at[c]], hbuf, s0)
        cp1 = pltpu.async_copy(ent_hbm.at[idx_v.at[N_CHUNKS + c]], tbuf, s1)
        cp2 = pltpu.async_copy(rn_hbm.at[idx_v.at[2 * N_CHUNKS + c]], rnbuf, s2)
        cp0.wait()
        cp1.wait()
        cp2.wait()

        def group(g, _):
            lane = lax.iota(jnp.int32, 16)
            acc = jnp.zeros((16,), jnp.float32)
            base = g * 16
            offs_h = idx_v[3 * N_CHUNKS + c, pl.ds(base, 16)]
            offs_t = idx_v[4 * N_CHUNKS + c, pl.ds(base, 16)]
            for i in range(16):
                s = base + i
                oh = offs_h[i]
                ot = offs_t[i]
                h = [hbuf[s, pl.ds(oh + 16 * j, 16)] for j in range(4)]
                t = [tbuf[s, pl.ds(ot + 16 * j, 16)] for j in range(4)]
                r = [rnbuf[s, pl.ds(16 * j, 16)] for j in range(4)]
                n = [rnbuf[s, pl.ds(64 + 16 * j, 16)] for j in range(4)]
                u = [h[j] - t[j] for j in range(4)]
                p = u[0] * n[0] + u[1] * n[1] + u[2] * n[2] + u[3] * n[3]
                d = _hsum(p, lane)
                df = [u[j] + r[j] - d * n[j] for j in range(4)]
                sq = df[0] * df[0] + df[1] * df[1] + df[2] * df[2] + df[3] * df[3]
                ss = _hsum(sq, lane)
                acc = jnp.where(lane == i, ss, acc)
            loss_v[pl.ds(c * CHUNK + g * 16, 16)] = _sqrt16(acc)
            return _

        lax.fori_loop(0, GROUPS, group, 0)

    pltpu.sync_copy(loss_v, out_hbm.at[pl.ds(wid * B_PER_W, B_PER_W)])


@jax.jit
def _transh(idx_pack, ent_t, rel_norm):
    ent2 = lax.optimization_barrier(_pair_table(ent_t))
    mesh = plsc.VectorSubcoreMesh(core_axis_name="c", subcore_axis_name="s")
    kfn = pl.kernel(
        _sc_body,
        out_type=jax.ShapeDtypeStruct((BATCH,), jnp.float32),
        mesh=mesh,
        scratch_types=[
            pltpu.VMEM((5 * N_CHUNKS, CHUNK), jnp.int32),
            pltpu.VMEM((CHUNK, 2 * K), jnp.float32),
            pltpu.VMEM((CHUNK, 2 * K), jnp.float32),
            pltpu.VMEM((CHUNK, 2 * K), jnp.float32),
            pltpu.VMEM((B_PER_W,), jnp.float32),
            pltpu.SemaphoreType.DMA,
            pltpu.SemaphoreType.DMA,
            pltpu.SemaphoreType.DMA,
        ],
    )
    return kfn(idx_pack, ent2, rel_norm)


def kernel(head, relation, tail, entity_emb, relation_emb, norm_emb):
    head = jnp.asarray(head, jnp.int32)
    tail = jnp.asarray(tail, jnp.int32)
    rel = jnp.asarray(relation, jnp.int32).reshape(NW, N_CHUNKS, CHUNK)
    h_line = jnp.where(head < SPLIT, head, head - SPLIT).reshape(NW, N_CHUNKS, CHUNK)
    t_line = jnp.where(tail < SPLIT, tail, tail - SPLIT).reshape(NW, N_CHUNKS, CHUNK)
    h_off = jnp.where(head < SPLIT, 0, K).astype(jnp.int32).reshape(NW, N_CHUNKS, CHUNK)
    t_off = jnp.where(tail < SPLIT, 0, K).astype(jnp.int32).reshape(NW, N_CHUNKS, CHUNK)
    idx_pack = jnp.concatenate([h_line, t_line, rel, h_off, t_off], axis=1)
    rel_norm = jnp.concatenate([relation_emb, norm_emb], axis=1)
    return _transh(idx_pack, entity_emb.T, rel_norm)
